# Initial kernel scaffold; baseline (speedup 1.0000x reference)
import jax
import jax.numpy as jnp
from jax.experimental import pallas as pl


def kernel(x, edge_index, edge_attr, W1, b1, gamma, beta, W2, b2):
    raise NotImplementedError("write your pallas kernel here")


# speedup vs baseline: 3.2270x; 3.2270x over previous
import functools

import jax
import jax.numpy as jnp
from jax import lax
from jax.experimental import pallas as pl
from jax.experimental.pallas import tpu as pltpu
import jax.experimental.pallas.tpu_sc as plsc

E = 320000
N = 10000
D = 128
DE = 16

NC = 2
NS = 16
NW = NC * NS
EW = E // NW
C = 80
CHUNKS = EW // C

_mesh = plsc.VectorSubcoreMesh(core_axis_name="c", subcore_axis_name="s",
                               num_cores=NC, num_subcores=NS)


@functools.partial(
    pl.kernel,
    out_type=(jax.ShapeDtypeStruct((E, D), jnp.float32),
              jax.ShapeDtypeStruct((E, D), jnp.float32)),
    mesh=_mesh,
    scratch_types=[
        pltpu.VMEM((C,), jnp.int32),
        pltpu.VMEM((C,), jnp.int32),
        pltpu.VMEM((C, D), jnp.float32),
        pltpu.VMEM((C, D), jnp.float32),
        pltpu.SemaphoreType.DMA,
        pltpu.SemaphoreType.DMA,
    ],
)
def _gather_kernel(x_hbm, ii_hbm, jj_hbm, xi_hbm, xj_hbm,
                   ii_v, jj_v, ri_v, rj_v, s1, s2):
    cid = lax.axis_index("c")
    sid = lax.axis_index("s")
    wid = cid * NS + sid
    base0 = wid * EW

    def body(k, carry):
        base = base0 + k * C
        pltpu.sync_copy(ii_hbm.at[pl.ds(base, C)], ii_v)
        pltpu.sync_copy(jj_hbm.at[pl.ds(base, C)], jj_v)
        a = pltpu.async_copy(x_hbm.at[ii_v], ri_v, s1)
        b = pltpu.async_copy(x_hbm.at[jj_v], rj_v, s2)
        a.wait()
        b.wait()
        pltpu.sync_copy(ri_v, xi_hbm.at[pl.ds(base, C)])
        pltpu.sync_copy(rj_v, xj_hbm.at[pl.ds(base, C)])
        return carry

    lax.fori_loop(0, CHUNKS, body, 0)


@functools.partial(
    pl.kernel,
    out_type=jax.ShapeDtypeStruct((NC, N, D), jnp.float32),
    mesh=_mesh,
    scratch_types=[
        pltpu.VMEM((C,), jnp.int32),
        pltpu.VMEM((C, D), jnp.float32),
        pltpu.VMEM_SHARED((N, D), jnp.float32),
    ],
)
def _scatter_kernel(m_hbm, jj_hbm, out_hbm, jj_v, rows_v, acc_sh):
    cid = lax.axis_index("c")
    sid = lax.axis_index("s")

    zero16 = jnp.zeros((16,), jnp.float32)

    def zrow(r, carry):
        for c8 in range(D // 16):
            rows_v[r, pl.ds(c8 * 16, 16)] = zero16
        return carry

    lax.fori_loop(0, C, zrow, 0)

    n_chunks = N // C
    per_tile = (n_chunks + NS - 1) // NS
    for k in range(per_tile):
        ch = sid * per_tile + k

        @pl.when(ch < n_chunks)
        def _():
            pltpu.sync_copy(rows_v, acc_sh.at[pl.ds(ch * C, C)])

    plsc.subcore_barrier()

    base0 = cid * (E // NC) + sid * EW

    def body(k, carry):
        base = base0 + k * C
        pltpu.sync_copy(jj_hbm.at[pl.ds(base, C)], jj_v)
        pltpu.sync_copy(m_hbm.at[pl.ds(base, C)], rows_v)
        pltpu.sync_copy(rows_v, acc_sh.at[jj_v], add=True)
        return carry

    lax.fori_loop(0, CHUNKS, body, 0)
    plsc.subcore_barrier()

    for k in range(per_tile):
        ch = sid * per_tile + k

        @pl.when(ch < n_chunks)
        def _():
            pltpu.sync_copy(acc_sh.at[pl.ds(ch * C, C)], rows_v)
            pltpu.sync_copy(rows_v, out_hbm.at[cid, pl.ds(ch * C, C)])


_EB = 3200

_GP = 0.3275911
_GA1 = 0.254829592
_GA2 = -0.284496736
_GA3 = 1.421413741
_GA4 = -1.453152027
_GA5 = 1.061405429


def _gelu_exact(x):
    z = jnp.abs(x) * 0.7071067811865476
    t = 1.0 / (1.0 + _GP * z)
    poly = t * (_GA1 + t * (_GA2 + t * (_GA3 + t * (_GA4 + t * _GA5))))
    e = 1.0 - poly * jnp.exp(-z * z)
    erf = jnp.where(x >= 0, e, -e)
    return 0.5 * x * (1.0 + erf)


def _mlp_body(xi_ref, xj_ref, ea_ref, w1a_ref, w1b_ref, w1c_ref, b1_ref,
              g_ref, be_ref, w2_ref, b2_ref, o_ref):
    h = jnp.dot(xi_ref[...], w1a_ref[...], preferred_element_type=jnp.float32)
    h = h + jnp.dot(xj_ref[...], w1b_ref[...], preferred_element_type=jnp.float32)
    h = h + jnp.dot(ea_ref[...], w1c_ref[...], preferred_element_type=jnp.float32)
    h = h + b1_ref[...]
    mean = jnp.mean(h, axis=1, keepdims=True)
    dlt = h - mean
    var = jnp.mean(dlt * dlt, axis=1, keepdims=True)
    hn = dlt * lax.rsqrt(var + 1e-5)
    hn = hn * g_ref[...] + be_ref[...]
    ge = _gelu_exact(hn)
    o_ref[...] = jnp.dot(ge, w2_ref[...],
                         preferred_element_type=jnp.float32) + b2_ref[...]


def _mlp(xi, xj, ea, w1aT, w1bT, w1cT, b1, gamma, beta, w2T, b2):
    nb = E // _EB
    row = lambda b: (b, 0)
    full = lambda b: (0, 0)
    return pl.pallas_call(
        _mlp_body,
        grid=(nb,),
        in_specs=[
            pl.BlockSpec((_EB, D), row),
            pl.BlockSpec((_EB, D), row),
            pl.BlockSpec((_EB, DE), row),
            pl.BlockSpec((D, D), full),
            pl.BlockSpec((D, D), full),
            pl.BlockSpec((DE, D), full),
            pl.BlockSpec((1, D), full),
            pl.BlockSpec((1, D), full),
            pl.BlockSpec((1, D), full),
            pl.BlockSpec((D, D), full),
            pl.BlockSpec((1, D), full),
        ],
        out_specs=pl.BlockSpec((_EB, D), row),
        out_shape=jax.ShapeDtypeStruct((E, D), jnp.float32),
    )(xi, xj, ea, w1aT, w1bT, w1cT, b1, gamma, beta, w2T, b2)


def _add_body(p_ref, o_ref):
    o_ref[...] = p_ref[0] + p_ref[1]


def _add_partials(p):
    nb = 10
    rb = N // nb
    return pl.pallas_call(
        _add_body,
        grid=(nb,),
        in_specs=[pl.BlockSpec((NC, rb, D), lambda b: (0, b, 0))],
        out_specs=pl.BlockSpec((rb, D), lambda b: (b, 0)),
        out_shape=jax.ShapeDtypeStruct((N, D), jnp.float32),
    )(p)


def kernel(x, edge_index, edge_attr, W1, b1, gamma, beta, W2, b2):
    ii = edge_index[0]
    jj = edge_index[1]
    w1aT = W1[:, :D].T
    w1bT = W1[:, D:2 * D].T
    w1cT = W1[:, 2 * D:].T
    xi, xj = _gather_kernel(x, ii, jj)
    m = _mlp(xi, xj, edge_attr, w1aT, w1bT, w1cT,
             b1[None, :], gamma[None, :], beta[None, :], W2.T, b2[None, :])
    parts = _scatter_kernel(m, jj)
    return _add_partials(parts)


# speedup vs baseline: 3.6224x; 1.1225x over previous
import functools

import jax
import jax.numpy as jnp
from jax import lax
from jax.experimental import pallas as pl
from jax.experimental.pallas import tpu as pltpu
import jax.experimental.pallas.tpu_sc as plsc

E = 320000
N = 10000
D = 128
DE = 16

NC = 2
NS = 16
NW = NC * NS
EW = E // NW
C = 40
CHUNKS = EW // C

_mesh = plsc.VectorSubcoreMesh(core_axis_name="c", subcore_axis_name="s",
                               num_cores=NC, num_subcores=NS)


@functools.partial(
    pl.kernel,
    out_type=(jax.ShapeDtypeStruct((E, D), jnp.float32),
              jax.ShapeDtypeStruct((E, D), jnp.float32)),
    mesh=_mesh,
    scratch_types=[
        pltpu.VMEM((EW,), jnp.int32),
        pltpu.VMEM((EW,), jnp.int32),
        pltpu.VMEM((2, C, D), jnp.float32),
        pltpu.VMEM((2, C, D), jnp.float32),
        pltpu.SemaphoreType.DMA((2,)),
        pltpu.SemaphoreType.DMA((2,)),
        pltpu.SemaphoreType.DMA((2,)),
        pltpu.SemaphoreType.DMA((2,)),
    ],
)
def _gather_kernel(x_hbm, ii_hbm, jj_hbm, xi_hbm, xj_hbm,
                   iib, jjb, ri, rj, sgi, sgj, swi, swj):
    cid = lax.axis_index("c")
    sid = lax.axis_index("s")
    wid = cid * NS + sid
    base0 = wid * EW

    pltpu.sync_copy(ii_hbm.at[pl.ds(base0, EW)], iib)
    pltpu.sync_copy(jj_hbm.at[pl.ds(base0, EW)], jjb)

    def start_gather(k, b):
        pltpu.async_copy(x_hbm.at[iib.at[pl.ds(k * C, C)]], ri.at[b], sgi.at[b])
        pltpu.async_copy(x_hbm.at[jjb.at[pl.ds(k * C, C)]], rj.at[b], sgj.at[b])

    def wait_gather(b):
        pltpu.make_async_copy(x_hbm.at[pl.ds(0, C)], ri.at[b], sgi.at[b]).wait()
        pltpu.make_async_copy(x_hbm.at[pl.ds(0, C)], rj.at[b], sgj.at[b]).wait()

    def start_writeout(k, b):
        pltpu.async_copy(ri.at[b], xi_hbm.at[pl.ds(base0 + k * C, C)], swi.at[b])
        pltpu.async_copy(rj.at[b], xj_hbm.at[pl.ds(base0 + k * C, C)], swj.at[b])

    def wait_writeout(b):
        pltpu.make_async_copy(ri.at[b], xi_hbm.at[pl.ds(0, C)], swi.at[b]).wait()
        pltpu.make_async_copy(rj.at[b], xj_hbm.at[pl.ds(0, C)], swj.at[b]).wait()

    start_gather(0, 0)

    def group(g, carry):
        for b in range(2):
            k = 2 * g + b
            wait_gather(b)
            start_writeout(k, b)

            @pl.when(k + 1 < CHUNKS)
            def _():
                @pl.when(k >= 1)
                def _():
                    wait_writeout(1 - b)

                start_gather(k + 1, 1 - b)
        return carry

    lax.fori_loop(0, CHUNKS // 2, group, 0)
    wait_writeout(0)
    wait_writeout(1)


@functools.partial(
    pl.kernel,
    out_type=jax.ShapeDtypeStruct((NC, N, D), jnp.float32),
    mesh=_mesh,
    scratch_types=[
        pltpu.VMEM((2, C), jnp.int32),
        pltpu.VMEM((2, C, D), jnp.float32),
        pltpu.VMEM((C, D), jnp.float32),
        pltpu.VMEM_SHARED((N, D), jnp.float32),
        pltpu.SemaphoreType.DMA((2,)),
        pltpu.SemaphoreType.DMA((2,)),
        pltpu.SemaphoreType.DMA((2,)),
    ],
)
def _scatter_kernel(m_hbm, jj_hbm, out_hbm, jjb, rows, stg, acc_sh,
                    sji, sri, ssc):
    cid = lax.axis_index("c")
    sid = lax.axis_index("s")

    zero16 = jnp.zeros((16,), jnp.float32)

    def zrow(r, carry):
        for c8 in range(D // 16):
            stg[r, pl.ds(c8 * 16, 16)] = zero16
        return carry

    lax.fori_loop(0, C, zrow, 0)

    n_zchunks = N // C
    per_tile = (n_zchunks + NS - 1) // NS
    for k in range(per_tile):
        ch = sid * per_tile + k

        @pl.when(ch < n_zchunks)
        def _():
            pltpu.sync_copy(stg, acc_sh.at[pl.ds(ch * C, C)])

    plsc.subcore_barrier()

    base0 = cid * (E // NC) + sid * EW

    def start_in(k, b):
        pltpu.async_copy(jj_hbm.at[pl.ds(base0 + k * C, C)], jjb.at[b], sji.at[b])
        pltpu.async_copy(m_hbm.at[pl.ds(base0 + k * C, C)], rows.at[b], sri.at[b])

    def wait_in(b):
        pltpu.make_async_copy(jj_hbm.at[pl.ds(0, C)], jjb.at[b], sji.at[b]).wait()
        pltpu.make_async_copy(m_hbm.at[pl.ds(0, C)], rows.at[b], sri.at[b]).wait()

    def start_scat(b):
        pltpu.async_copy(rows.at[b], acc_sh.at[jjb.at[b]], ssc.at[b], add=True)

    def wait_scat(b):
        pltpu.make_async_copy(rows.at[b], acc_sh.at[pl.ds(0, C)], ssc.at[b]).wait()

    start_in(0, 0)

    def group(g, carry):
        for b in range(2):
            k = 2 * g + b
            wait_in(b)
            start_scat(b)

            @pl.when(k + 1 < CHUNKS)
            def _():
                @pl.when(k >= 1)
                def _():
                    wait_scat(1 - b)

                start_in(k + 1, 1 - b)
        return carry

    lax.fori_loop(0, CHUNKS // 2, group, 0)
    wait_scat(0)
    wait_scat(1)
    plsc.subcore_barrier()

    for k in range(per_tile):
        ch = sid * per_tile + k

        @pl.when(ch < n_zchunks)
        def _():
            pltpu.sync_copy(acc_sh.at[pl.ds(ch * C, C)], stg)
            pltpu.sync_copy(stg, out_hbm.at[cid, pl.ds(ch * C, C)])


_EB = 3200

_GP = 0.3275911
_GA1 = 0.254829592
_GA2 = -0.284496736
_GA3 = 1.421413741
_GA4 = -1.453152027
_GA5 = 1.061405429


def _gelu_exact(x):
    z = jnp.abs(x) * 0.7071067811865476
    t = 1.0 / (1.0 + _GP * z)
    poly = t * (_GA1 + t * (_GA2 + t * (_GA3 + t * (_GA4 + t * _GA5))))
    e = 1.0 - poly * jnp.exp(-z * z)
    erf = jnp.where(x >= 0, e, -e)
    return 0.5 * x * (1.0 + erf)


def _mlp_body(xi_ref, xj_ref, ea_ref, w1a_ref, w1b_ref, w1c_ref, b1_ref,
              g_ref, be_ref, w2_ref, b2_ref, o_ref):
    h = jnp.dot(xi_ref[...], w1a_ref[...], preferred_element_type=jnp.float32)
    h = h + jnp.dot(xj_ref[...], w1b_ref[...], preferred_element_type=jnp.float32)
    h = h + jnp.dot(ea_ref[...], w1c_ref[...], preferred_element_type=jnp.float32)
    h = h + b1_ref[...]
    mean = jnp.mean(h, axis=1, keepdims=True)
    dlt = h - mean
    var = jnp.mean(dlt * dlt, axis=1, keepdims=True)
    hn = dlt * lax.rsqrt(var + 1e-5)
    hn = hn * g_ref[...] + be_ref[...]
    ge = _gelu_exact(hn)
    o_ref[...] = jnp.dot(ge, w2_ref[...],
                         preferred_element_type=jnp.float32) + b2_ref[...]


def _mlp(xi, xj, ea, w1aT, w1bT, w1cT, b1, gamma, beta, w2T, b2):
    nb = E // _EB
    row = lambda b: (b, 0)
    full = lambda b: (0, 0)
    return pl.pallas_call(
        _mlp_body,
        grid=(nb,),
        in_specs=[
            pl.BlockSpec((_EB, D), row),
            pl.BlockSpec((_EB, D), row),
            pl.BlockSpec((_EB, DE), row),
            pl.BlockSpec((D, D), full),
            pl.BlockSpec((D, D), full),
            pl.BlockSpec((DE, D), full),
            pl.BlockSpec((1, D), full),
            pl.BlockSpec((1, D), full),
            pl.BlockSpec((1, D), full),
            pl.BlockSpec((D, D), full),
            pl.BlockSpec((1, D), full),
        ],
        out_specs=pl.BlockSpec((_EB, D), row),
        out_shape=jax.ShapeDtypeStruct((E, D), jnp.float32),
    )(xi, xj, ea, w1aT, w1bT, w1cT, b1, gamma, beta, w2T, b2)


def _add_body(p_ref, o_ref):
    o_ref[...] = p_ref[0] + p_ref[1]


def _add_partials(p):
    nb = 10
    rb = N // nb
    return pl.pallas_call(
        _add_body,
        grid=(nb,),
        in_specs=[pl.BlockSpec((NC, rb, D), lambda b: (0, b, 0))],
        out_specs=pl.BlockSpec((rb, D), lambda b: (b, 0)),
        out_shape=jax.ShapeDtypeStruct((N, D), jnp.float32),
    )(p)


def kernel(x, edge_index, edge_attr, W1, b1, gamma, beta, W2, b2):
    ii = edge_index[0]
    jj = edge_index[1]
    w1aT = W1[:, :D].T
    w1bT = W1[:, D:2 * D].T
    w1cT = W1[:, 2 * D:].T
    xi, xj = _gather_kernel(x, ii, jj)
    m = _mlp(xi, xj, edge_attr, w1aT, w1bT, w1cT,
             b1[None, :], gamma[None, :], beta[None, :], W2.T, b2[None, :])
    parts = _scatter_kernel(m, jj)
    return _add_partials(parts)


# speedup vs baseline: 3.9651x; 1.0946x over previous
import functools

import jax
import jax.numpy as jnp
from jax import lax
from jax.experimental import pallas as pl
from jax.experimental.pallas import tpu as pltpu
import jax.experimental.pallas.tpu_sc as plsc

E = 320000
N = 10000
D = 128
DE = 16

NC = 2
NS = 16
NW = NC * NS
EW = E // NW
C = 40
CHUNKS = EW // C
ZC = 80
NZ = N // ZC
ZPT = (NZ + NS - 1) // NS

_mesh = plsc.VectorSubcoreMesh(core_axis_name="c", subcore_axis_name="s",
                               num_cores=NC, num_subcores=NS)


@functools.partial(
    pl.kernel,
    out_type=jax.ShapeDtypeStruct((E, D), jnp.float32),
    mesh=_mesh,
    scratch_types=[
        pltpu.VMEM((EW,), jnp.int32),
        pltpu.VMEM((EW,), jnp.int32),
        pltpu.VMEM((2, C, D), jnp.float32),
        pltpu.VMEM((2, C, D), jnp.float32),
        pltpu.SemaphoreType.DMA((2,)),
        pltpu.SemaphoreType.DMA((2,)),
        pltpu.SemaphoreType.DMA((2,)),
    ],
)
def _gather_kernel(u_hbm, v_hbm, ii_hbm, jj_hbm, g_hbm,
                   iib, jjb, ru, rv, sgu, sgv, sw):
    cid = lax.axis_index("c")
    sid = lax.axis_index("s")
    wid = cid * NS + sid
    base0 = wid * EW

    pltpu.sync_copy(ii_hbm.at[pl.ds(base0, EW)], iib)
    pltpu.sync_copy(jj_hbm.at[pl.ds(base0, EW)], jjb)

    def start_gather(k, b):
        pltpu.async_copy(u_hbm.at[iib.at[pl.ds(k * C, C)]], ru.at[b], sgu.at[b])
        pltpu.async_copy(v_hbm.at[jjb.at[pl.ds(k * C, C)]], rv.at[b], sgv.at[b])

    def wait_gather(b):
        pltpu.make_async_copy(u_hbm.at[pl.ds(0, C)], ru.at[b], sgu.at[b]).wait()
        pltpu.make_async_copy(v_hbm.at[pl.ds(0, C)], rv.at[b], sgv.at[b]).wait()

    def add_rows(b):
        @plsc.parallel_loop(0, C, unroll=4)
        def _(e):
            for c8 in range(D // 16):
                sl = pl.ds(c8 * 16, 16)
                ru[b, e, sl] = ru[b, e, sl] + rv[b, e, sl]

    def start_writeout(k, b):
        pltpu.async_copy(ru.at[b], g_hbm.at[pl.ds(base0 + k * C, C)], sw.at[b])

    def wait_writeout(b):
        pltpu.make_async_copy(ru.at[b], g_hbm.at[pl.ds(0, C)], sw.at[b]).wait()

    start_gather(0, 0)

    def group(g, carry):
        for b in range(2):
            k = 2 * g + b
            wait_gather(b)

            @pl.when(k + 1 < CHUNKS)
            def _():
                @pl.when(k >= 1)
                def _():
                    wait_writeout(1 - b)

                start_gather(k + 1, 1 - b)

            add_rows(b)
            start_writeout(k, b)
        return carry

    lax.fori_loop(0, CHUNKS // 2, group, 0)
    wait_writeout(0)
    wait_writeout(1)


@functools.partial(
    pl.kernel,
    out_type=jax.ShapeDtypeStruct((NC, N, D), jnp.float32),
    mesh=_mesh,
    scratch_types=[
        pltpu.VMEM((2, C), jnp.int32),
        pltpu.VMEM((2, C, D), jnp.float32),
        pltpu.VMEM((ZC, D), jnp.float32),
        pltpu.VMEM_SHARED((N, D), jnp.float32),
        pltpu.SemaphoreType.DMA((2,)),
        pltpu.SemaphoreType.DMA((2,)),
        pltpu.SemaphoreType.DMA((2,)),
    ],
)
def _scatter_kernel(m_hbm, jj_hbm, out_hbm, jjb, rows, stg, acc_sh,
                    sji, sri, ssc):
    cid = lax.axis_index("c")
    sid = lax.axis_index("s")

    zero16 = jnp.zeros((16,), jnp.float32)

    @plsc.parallel_loop(0, ZC, unroll=4)
    def _(r):
        for c8 in range(D // 16):
            stg[r, pl.ds(c8 * 16, 16)] = zero16

    for k in range(ZPT):
        ch = sid * ZPT + k

        @pl.when(ch < NZ)
        def _():
            pltpu.sync_copy(stg, acc_sh.at[pl.ds(ch * ZC, ZC)])

    plsc.subcore_barrier()

    base0 = cid * (E // NC) + sid * EW

    def start_in(k, b):
        pltpu.async_copy(jj_hbm.at[pl.ds(base0 + k * C, C)], jjb.at[b], sji.at[b])
        pltpu.async_copy(m_hbm.at[pl.ds(base0 + k * C, C)], rows.at[b], sri.at[b])

    def wait_in(b):
        pltpu.make_async_copy(jj_hbm.at[pl.ds(0, C)], jjb.at[b], sji.at[b]).wait()
        pltpu.make_async_copy(m_hbm.at[pl.ds(0, C)], rows.at[b], sri.at[b]).wait()

    def start_scat(b):
        pltpu.async_copy(rows.at[b], acc_sh.at[jjb.at[b]], ssc.at[b], add=True)

    def wait_scat(b):
        pltpu.make_async_copy(rows.at[b], acc_sh.at[pl.ds(0, C)], ssc.at[b]).wait()

    start_in(0, 0)

    def group(g, carry):
        for b in range(2):
            k = 2 * g + b
            wait_in(b)
            start_scat(b)

            @pl.when(k + 1 < CHUNKS)
            def _():
                @pl.when(k >= 1)
                def _():
                    wait_scat(1 - b)

                start_in(k + 1, 1 - b)
        return carry

    lax.fori_loop(0, CHUNKS // 2, group, 0)
    wait_scat(0)
    wait_scat(1)
    plsc.subcore_barrier()

    for k in range(ZPT):
        ch = sid * ZPT + k

        @pl.when(ch < NZ)
        def _():
            pltpu.sync_copy(acc_sh.at[pl.ds(ch * ZC, ZC)], stg)
            pltpu.sync_copy(stg, out_hbm.at[cid, pl.ds(ch * ZC, ZC)])


def _uv_body(x_ref, w1a_ref, w1b_ref, u_ref, v_ref):
    u_ref[...] = jnp.dot(x_ref[...], w1a_ref[...],
                         preferred_element_type=jnp.float32)
    v_ref[...] = jnp.dot(x_ref[...], w1b_ref[...],
                         preferred_element_type=jnp.float32)


def _uv(x, w1aT, w1bT):
    nb = 10
    rb = N // nb
    row = lambda b: (b, 0)
    full = lambda b: (0, 0)
    return pl.pallas_call(
        _uv_body,
        grid=(nb,),
        in_specs=[
            pl.BlockSpec((rb, D), row),
            pl.BlockSpec((D, D), full),
            pl.BlockSpec((D, D), full),
        ],
        out_specs=(pl.BlockSpec((rb, D), row), pl.BlockSpec((rb, D), row)),
        out_shape=(jax.ShapeDtypeStruct((N, D), jnp.float32),
                   jax.ShapeDtypeStruct((N, D), jnp.float32)),
    )(x, w1aT, w1bT)


_EB = 3200

_GP = 0.3275911
_GA1 = 0.254829592
_GA2 = -0.284496736
_GA3 = 1.421413741
_GA4 = -1.453152027
_GA5 = 1.061405429


def _gelu_exact(x):
    z = jnp.abs(x) * 0.7071067811865476
    t = 1.0 / (1.0 + _GP * z)
    poly = t * (_GA1 + t * (_GA2 + t * (_GA3 + t * (_GA4 + t * _GA5))))
    e = 1.0 - poly * jnp.exp(-z * z)
    erf = jnp.where(x >= 0, e, -e)
    return 0.5 * x * (1.0 + erf)


def _mlp_body(g_ref, ea_ref, w1c_ref, b1_ref, gm_ref, be_ref, w2_ref, b2_ref,
              o_ref):
    h = g_ref[...] + jnp.dot(ea_ref[...], w1c_ref[...],
                             preferred_element_type=jnp.float32)
    h = h + b1_ref[...]
    mean = jnp.mean(h, axis=1, keepdims=True)
    dlt = h - mean
    var = jnp.mean(dlt * dlt, axis=1, keepdims=True)
    hn = dlt * lax.rsqrt(var + 1e-5)
    hn = hn * gm_ref[...] + be_ref[...]
    ge = _gelu_exact(hn)
    o_ref[...] = jnp.dot(ge, w2_ref[...],
                         preferred_element_type=jnp.float32) + b2_ref[...]


def _mlp(g, ea, w1cT, b1, gamma, beta, w2T, b2):
    nb = E // _EB
    row = lambda b: (b, 0)
    full = lambda b: (0, 0)
    return pl.pallas_call(
        _mlp_body,
        grid=(nb,),
        in_specs=[
            pl.BlockSpec((_EB, D), row),
            pl.BlockSpec((_EB, DE), row),
            pl.BlockSpec((DE, D), full),
            pl.BlockSpec((1, D), full),
            pl.BlockSpec((1, D), full),
            pl.BlockSpec((1, D), full),
            pl.BlockSpec((D, D), full),
            pl.BlockSpec((1, D), full),
        ],
        out_specs=pl.BlockSpec((_EB, D), row),
        out_shape=jax.ShapeDtypeStruct((E, D), jnp.float32),
    )(g, ea, w1cT, b1, gamma, beta, w2T, b2)


def _add_body(p_ref, o_ref):
    o_ref[...] = p_ref[0] + p_ref[1]


def _add_partials(p):
    nb = 10
    rb = N // nb
    return pl.pallas_call(
        _add_body,
        grid=(nb,),
        in_specs=[pl.BlockSpec((NC, rb, D), lambda b: (0, b, 0))],
        out_specs=pl.BlockSpec((rb, D), lambda b: (b, 0)),
        out_shape=jax.ShapeDtypeStruct((N, D), jnp.float32),
    )(p)


def kernel(x, edge_index, edge_attr, W1, b1, gamma, beta, W2, b2):
    ii = edge_index[0]
    jj = edge_index[1]
    w1aT = W1[:, :D].T
    w1bT = W1[:, D:2 * D].T
    w1cT = W1[:, 2 * D:].T
    u, v = _uv(x, w1aT, w1bT)
    g = _gather_kernel(u, v, ii, jj)
    m = _mlp(g, edge_attr, w1cT,
             b1[None, :], gamma[None, :], beta[None, :], W2.T, b2[None, :])
    parts = _scatter_kernel(m, jj)
    return _add_partials(parts)


# speedup vs baseline: 4.6916x; 1.1832x over previous
import functools

import jax
import jax.numpy as jnp
from jax import lax
from jax.experimental import pallas as pl
from jax.experimental.pallas import tpu as pltpu
import jax.experimental.pallas.tpu_sc as plsc

E = 320000
N = 10000
D = 128
DE = 16

SC_SPLIT = 2
ES = E // SC_SPLIT

NC = 2
NS = 16
NW = NC * NS
C = 40
ZC = 80
NZ = N // ZC
ZPT = (NZ + NS - 1) // NS

_mesh = plsc.VectorSubcoreMesh(core_axis_name="c", subcore_axis_name="s",
                               num_cores=NC, num_subcores=NS)


def _make_gather(ne):
    ew = ne // NW
    chunks = ew // C
    groups = chunks // 2
    has_tail = chunks % 2 == 1

    @functools.partial(
        pl.kernel,
        out_type=jax.ShapeDtypeStruct((ne, D), jnp.float32),
        mesh=_mesh,
        scratch_types=[
            pltpu.VMEM((ew,), jnp.int32),
            pltpu.VMEM((ew,), jnp.int32),
            pltpu.VMEM((2, C, D), jnp.float32),
            pltpu.VMEM((2, C, D), jnp.float32),
            pltpu.SemaphoreType.DMA((2,)),
            pltpu.SemaphoreType.DMA((2,)),
            pltpu.SemaphoreType.DMA((2,)),
        ],
    )
    def gather_kernel(u_hbm, v_hbm, ii_hbm, jj_hbm, g_hbm,
                      iib, jjb, ru, rv, sgu, sgv, sw):
        cid = lax.axis_index("c")
        sid = lax.axis_index("s")
        wid = cid * NS + sid
        base0 = wid * ew

        pltpu.sync_copy(ii_hbm.at[pl.ds(base0, ew)], iib)
        pltpu.sync_copy(jj_hbm.at[pl.ds(base0, ew)], jjb)

        def start_gather(k, b):
            pltpu.async_copy(u_hbm.at[iib.at[pl.ds(k * C, C)]], ru.at[b],
                             sgu.at[b])
            pltpu.async_copy(v_hbm.at[jjb.at[pl.ds(k * C, C)]], rv.at[b],
                             sgv.at[b])

        def wait_gather(b):
            pltpu.make_async_copy(u_hbm.at[pl.ds(0, C)], ru.at[b],
                                  sgu.at[b]).wait()
            pltpu.make_async_copy(v_hbm.at[pl.ds(0, C)], rv.at[b],
                                  sgv.at[b]).wait()

        def add_rows(b):
            @plsc.parallel_loop(0, C, unroll=4)
            def _(e):
                for c8 in range(D // 16):
                    sl = pl.ds(c8 * 16, 16)
                    ru[b, e, sl] = ru[b, e, sl] + rv[b, e, sl]

        def start_writeout(k, b):
            pltpu.async_copy(ru.at[b], g_hbm.at[pl.ds(base0 + k * C, C)],
                             sw.at[b])

        def wait_writeout(b):
            pltpu.make_async_copy(ru.at[b], g_hbm.at[pl.ds(0, C)],
                                  sw.at[b]).wait()

        start_gather(0, 0)

        def group(g, carry):
            for b in range(2):
                k = 2 * g + b
                wait_gather(b)

                @pl.when(k + 1 < chunks)
                def _():
                    @pl.when(k >= 1)
                    def _():
                        wait_writeout(1 - b)

                    start_gather(k + 1, 1 - b)

                add_rows(b)
                start_writeout(k, b)
            return carry

        lax.fori_loop(0, groups, group, 0)
        if has_tail:
            wait_gather(0)
            add_rows(0)
            start_writeout(chunks - 1, 0)
        wait_writeout(1)
        wait_writeout(0)

    return gather_kernel


def _make_scatter(ne):
    ew = ne // NW
    chunks = ew // C
    groups = chunks // 2
    has_tail = chunks % 2 == 1

    @functools.partial(
        pl.kernel,
        out_type=jax.ShapeDtypeStruct((NC, N, D), jnp.float32),
        mesh=_mesh,
        scratch_types=[
            pltpu.VMEM((2, C), jnp.int32),
            pltpu.VMEM((2, C, D), jnp.float32),
            pltpu.VMEM((ZC, D), jnp.float32),
            pltpu.VMEM_SHARED((N, D), jnp.float32),
            pltpu.SemaphoreType.DMA((2,)),
            pltpu.SemaphoreType.DMA((2,)),
            pltpu.SemaphoreType.DMA((2,)),
        ],
    )
    def scatter_kernel(m_hbm, jj_hbm, out_hbm, jjb, rows, stg, acc_sh,
                       sji, sri, ssc):
        cid = lax.axis_index("c")
        sid = lax.axis_index("s")

        zero16 = jnp.zeros((16,), jnp.float32)

        @plsc.parallel_loop(0, ZC, unroll=4)
        def _(r):
            for c8 in range(D // 16):
                stg[r, pl.ds(c8 * 16, 16)] = zero16

        for k in range(ZPT):
            ch = sid * ZPT + k

            @pl.when(ch < NZ)
            def _():
                pltpu.sync_copy(stg, acc_sh.at[pl.ds(ch * ZC, ZC)])

        plsc.subcore_barrier()

        base0 = cid * (ne // NC) + sid * ew

        def start_in(k, b):
            pltpu.async_copy(jj_hbm.at[pl.ds(base0 + k * C, C)], jjb.at[b],
                             sji.at[b])
            pltpu.async_copy(m_hbm.at[pl.ds(base0 + k * C, C)], rows.at[b],
                             sri.at[b])

        def wait_in(b):
            pltpu.make_async_copy(jj_hbm.at[pl.ds(0, C)], jjb.at[b],
                                  sji.at[b]).wait()
            pltpu.make_async_copy(m_hbm.at[pl.ds(0, C)], rows.at[b],
                                  sri.at[b]).wait()

        def start_scat(b):
            pltpu.async_copy(rows.at[b], acc_sh.at[jjb.at[b]], ssc.at[b],
                             add=True)

        def wait_scat(b):
            pltpu.make_async_copy(rows.at[b], acc_sh.at[pl.ds(0, C)],
                                  ssc.at[b]).wait()

        start_in(0, 0)

        def group(g, carry):
            for b in range(2):
                k = 2 * g + b
                wait_in(b)
                start_scat(b)

                @pl.when(k + 1 < chunks)
                def _():
                    @pl.when(k >= 1)
                    def _():
                        wait_scat(1 - b)

                    start_in(k + 1, 1 - b)
            return carry

        lax.fori_loop(0, groups, group, 0)
        if has_tail:
            wait_in(0)
            start_scat(0)
        wait_scat(1)
        wait_scat(0)
        plsc.subcore_barrier()

        for k in range(ZPT):
            ch = sid * ZPT + k

            @pl.when(ch < NZ)
            def _():
                pltpu.sync_copy(acc_sh.at[pl.ds(ch * ZC, ZC)], stg)
                pltpu.sync_copy(stg, out_hbm.at[cid, pl.ds(ch * ZC, ZC)])

    return scatter_kernel


_gather_sc = _make_gather(ES)
_scatter_sc = _make_scatter(ES)


def _uv_body(x_ref, w1a_ref, w1b_ref, u_ref, v_ref):
    u_ref[...] = jnp.dot(x_ref[...], w1a_ref[...],
                         preferred_element_type=jnp.float32)
    v_ref[...] = jnp.dot(x_ref[...], w1b_ref[...],
                         preferred_element_type=jnp.float32)


def _uv(x, w1aT, w1bT):
    nb = 10
    rb = N // nb
    row = lambda b: (b, 0)
    full = lambda b: (0, 0)
    return pl.pallas_call(
        _uv_body,
        grid=(nb,),
        in_specs=[
            pl.BlockSpec((rb, D), row),
            pl.BlockSpec((D, D), full),
            pl.BlockSpec((D, D), full),
        ],
        out_specs=(pl.BlockSpec((rb, D), row), pl.BlockSpec((rb, D), row)),
        out_shape=(jax.ShapeDtypeStruct((N, D), jnp.float32),
                   jax.ShapeDtypeStruct((N, D), jnp.float32)),
    )(x, w1aT, w1bT)


_EB = 3200

_GP = 0.3275911
_GA1 = 0.254829592
_GA2 = -0.284496736
_GA3 = 1.421413741
_GA4 = -1.453152027
_GA5 = 1.061405429


def _gelu_exact(x):
    z = jnp.abs(x) * 0.7071067811865476
    t = 1.0 / (1.0 + _GP * z)
    poly = t * (_GA1 + t * (_GA2 + t * (_GA3 + t * (_GA4 + t * _GA5))))
    e = 1.0 - poly * jnp.exp(-z * z)
    erf = jnp.where(x >= 0, e, -e)
    return 0.5 * x * (1.0 + erf)


def _mlp_body(g_ref, ea_ref, w1c_ref, b1_ref, gm_ref, be_ref, w2_ref, b2_ref,
              o_ref):
    h = g_ref[...] + jnp.dot(ea_ref[...], w1c_ref[...],
                             preferred_element_type=jnp.float32)
    h = h + b1_ref[...]
    mean = jnp.mean(h, axis=1, keepdims=True)
    dlt = h - mean
    var = jnp.mean(dlt * dlt, axis=1, keepdims=True)
    hn = dlt * lax.rsqrt(var + 1e-5)
    hn = hn * gm_ref[...] + be_ref[...]
    ge = _gelu_exact(hn)
    o_ref[...] = jnp.dot(ge, w2_ref[...],
                         preferred_element_type=jnp.float32) + b2_ref[...]


def _mlp(g, ea, w1cT, b1, gamma, beta, w2T, b2):
    ne = g.shape[0]
    nb = ne // _EB
    row = lambda b: (b, 0)
    full = lambda b: (0, 0)
    return pl.pallas_call(
        _mlp_body,
        grid=(nb,),
        in_specs=[
            pl.BlockSpec((_EB, D), row),
            pl.BlockSpec((_EB, DE), row),
            pl.BlockSpec((DE, D), full),
            pl.BlockSpec((1, D), full),
            pl.BlockSpec((1, D), full),
            pl.BlockSpec((1, D), full),
            pl.BlockSpec((D, D), full),
            pl.BlockSpec((1, D), full),
        ],
        out_specs=pl.BlockSpec((_EB, D), row),
        out_shape=jax.ShapeDtypeStruct((ne, D), jnp.float32),
    )(g, ea, w1cT, b1, gamma, beta, w2T, b2)


def _add_body(p1_ref, p2_ref, o_ref):
    o_ref[...] = (p1_ref[0] + p1_ref[1]) + (p2_ref[0] + p2_ref[1])


def _add_partials(p1, p2):
    nb = 10
    rb = N // nb
    spec = pl.BlockSpec((NC, rb, D), lambda b: (0, b, 0))
    return pl.pallas_call(
        _add_body,
        grid=(nb,),
        in_specs=[spec, spec],
        out_specs=pl.BlockSpec((rb, D), lambda b: (b, 0)),
        out_shape=jax.ShapeDtypeStruct((N, D), jnp.float32),
    )(p1, p2)


def kernel(x, edge_index, edge_attr, W1, b1, gamma, beta, W2, b2):
    ii = edge_index[0]
    jj = edge_index[1]
    w1aT = W1[:, :D].T
    w1bT = W1[:, D:2 * D].T
    w1cT = W1[:, 2 * D:].T
    b1r = b1[None, :]
    gammar = gamma[None, :]
    betar = beta[None, :]
    w2T = W2.T
    b2r = b2[None, :]

    u, v = _uv(x, w1aT, w1bT)
    parts = []
    for h in range(SC_SPLIT):
        sl = slice(h * ES, (h + 1) * ES)
        g = _gather_sc(u, v, ii[sl], jj[sl])
        m = _mlp(g, edge_attr[sl], w1cT, b1r, gammar, betar, w2T, b2r)
        parts.append(_scatter_sc(m, jj[sl]))
    return _add_partials(*parts)


# speedup vs baseline: 5.5203x; 1.1767x over previous
import functools

import jax
import jax.numpy as jnp
from jax import lax
from jax.experimental import pallas as pl
from jax.experimental.pallas import tpu as pltpu
import jax.experimental.pallas.tpu_sc as plsc

E = 320000
N = 10000
D = 128
DE = 16

SC_SPLIT = 2
ES = E // SC_SPLIT

NC = 2
NS = 16
NW = NC * NS
C = 40
ZC = 80
NZ = N // ZC
ZPT = (NZ + NS - 1) // NS

_mesh = plsc.VectorSubcoreMesh(core_axis_name="c", subcore_axis_name="s",
                               num_cores=NC, num_subcores=NS)


def _make_gather(ne):
    ew = ne // NW
    chunks = ew // C
    nb4 = 4
    groups = chunks // nb4
    tail = chunks % nb4

    @functools.partial(
        pl.kernel,
        out_type=jax.ShapeDtypeStruct((ne, D), jnp.float32),
        mesh=_mesh,
        scratch_types=[
            pltpu.VMEM((ew,), jnp.int32),
            pltpu.VMEM((ew,), jnp.int32),
            pltpu.VMEM((nb4, C, D), jnp.float32),
            pltpu.VMEM((nb4, C, D), jnp.float32),
            pltpu.SemaphoreType.DMA((nb4,)),
            pltpu.SemaphoreType.DMA((nb4,)),
            pltpu.SemaphoreType.DMA((nb4,)),
        ],
    )
    def gather_kernel(u_hbm, v_hbm, ii_hbm, jj_hbm, g_hbm,
                      iib, jjb, ru, rv, sgu, sgv, sw):
        cid = lax.axis_index("c")
        sid = lax.axis_index("s")
        wid = cid * NS + sid
        base0 = wid * ew

        pltpu.sync_copy(ii_hbm.at[pl.ds(base0, ew)], iib)
        pltpu.sync_copy(jj_hbm.at[pl.ds(base0, ew)], jjb)

        def start_gather(k, b):
            pltpu.async_copy(u_hbm.at[iib.at[pl.ds(k * C, C)]], ru.at[b],
                             sgu.at[b])
            pltpu.async_copy(v_hbm.at[jjb.at[pl.ds(k * C, C)]], rv.at[b],
                             sgv.at[b])

        def wait_gather(b):
            pltpu.make_async_copy(u_hbm.at[pl.ds(0, C)], ru.at[b],
                                  sgu.at[b]).wait()
            pltpu.make_async_copy(v_hbm.at[pl.ds(0, C)], rv.at[b],
                                  sgv.at[b]).wait()

        def add_rows(b):
            @plsc.parallel_loop(0, C, unroll=8)
            def _(e):
                for c8 in range(D // 16):
                    sl = pl.ds(c8 * 16, 16)
                    ru[b, e, sl] = ru[b, e, sl] + rv[b, e, sl]

        def start_writeout(k, b):
            pltpu.async_copy(ru.at[b], g_hbm.at[pl.ds(base0 + k * C, C)],
                             sw.at[b])

        def wait_writeout(b):
            pltpu.make_async_copy(ru.at[b], g_hbm.at[pl.ds(0, C)],
                                  sw.at[b]).wait()

        for b in range(nb4 - 1):
            start_gather(b, b)

        def group(g, carry):
            for b in range(nb4):
                k = nb4 * g + b
                wait_gather(b)

                @pl.when(k + nb4 - 1 < chunks)
                def _():
                    @pl.when(k >= 1)
                    def _():
                        wait_writeout((b + nb4 - 1) % nb4)

                    start_gather(k + nb4 - 1, (b + nb4 - 1) % nb4)

                add_rows(b)
                start_writeout(k, b)
            return carry

        lax.fori_loop(0, groups, group, 0)
        for t in range(tail):
            k = groups * nb4 + t
            b = k % nb4
            wait_gather(b)
            add_rows(b)
            start_writeout(k, b)
        for b in range(nb4):
            wait_writeout(b)

    return gather_kernel


def _make_scatter(ne):
    ew = ne // NW
    chunks = ew // C
    nb4 = 4
    groups = chunks // nb4
    tail = chunks % nb4

    @functools.partial(
        pl.kernel,
        out_type=jax.ShapeDtypeStruct((NC, N, D), jnp.float32),
        mesh=_mesh,
        scratch_types=[
            pltpu.VMEM((nb4, C), jnp.int32),
            pltpu.VMEM((nb4, C, D), jnp.float32),
            pltpu.VMEM((ZC, D), jnp.float32),
            pltpu.VMEM_SHARED((N, D), jnp.float32),
            pltpu.SemaphoreType.DMA((nb4,)),
            pltpu.SemaphoreType.DMA((nb4,)),
            pltpu.SemaphoreType.DMA((nb4,)),
        ],
    )
    def scatter_kernel(m_hbm, jj_hbm, out_hbm, jjb, rows, stg, acc_sh,
                       sji, sri, ssc):
        cid = lax.axis_index("c")
        sid = lax.axis_index("s")

        zero16 = jnp.zeros((16,), jnp.float32)

        @plsc.parallel_loop(0, ZC, unroll=4)
        def _(r):
            for c8 in range(D // 16):
                stg[r, pl.ds(c8 * 16, 16)] = zero16

        for k in range(ZPT):
            ch = sid * ZPT + k

            @pl.when(ch < NZ)
            def _():
                pltpu.sync_copy(stg, acc_sh.at[pl.ds(ch * ZC, ZC)])

        plsc.subcore_barrier()

        base0 = cid * (ne // NC) + sid * ew

        def start_in(k, b):
            pltpu.async_copy(jj_hbm.at[pl.ds(base0 + k * C, C)], jjb.at[b],
                             sji.at[b])
            pltpu.async_copy(m_hbm.at[pl.ds(base0 + k * C, C)], rows.at[b],
                             sri.at[b])

        def wait_in(b):
            pltpu.make_async_copy(jj_hbm.at[pl.ds(0, C)], jjb.at[b],
                                  sji.at[b]).wait()
            pltpu.make_async_copy(m_hbm.at[pl.ds(0, C)], rows.at[b],
                                  sri.at[b]).wait()

        def start_scat(b):
            pltpu.async_copy(rows.at[b], acc_sh.at[jjb.at[b]], ssc.at[b],
                             add=True)

        def wait_scat(b):
            pltpu.make_async_copy(rows.at[b], acc_sh.at[pl.ds(0, C)],
                                  ssc.at[b]).wait()

        for b in range(nb4 - 1):
            start_in(b, b)

        def group(g, carry):
            for b in range(nb4):
                k = nb4 * g + b
                wait_in(b)
                start_scat(b)

                @pl.when(k + nb4 - 1 < chunks)
                def _():
                    @pl.when(k >= 1)
                    def _():
                        wait_scat((b + nb4 - 1) % nb4)

                    start_in(k + nb4 - 1, (b + nb4 - 1) % nb4)
            return carry

        lax.fori_loop(0, groups, group, 0)
        for t in range(tail):
            k = groups * nb4 + t
            b = k % nb4
            wait_in(b)
            start_scat(b)
        for b in range(nb4):
            wait_scat(b)
        plsc.subcore_barrier()

        for k in range(ZPT):
            ch = sid * ZPT + k

            @pl.when(ch < NZ)
            def _():
                pltpu.sync_copy(acc_sh.at[pl.ds(ch * ZC, ZC)], stg)
                pltpu.sync_copy(stg, out_hbm.at[cid, pl.ds(ch * ZC, ZC)])

    return scatter_kernel


_gather_sc = _make_gather(ES)
_scatter_sc = _make_scatter(ES)


def _uv_body(x_ref, w1a_ref, w1b_ref, u_ref, v_ref):
    u_ref[...] = jnp.dot(x_ref[...], w1a_ref[...],
                         preferred_element_type=jnp.float32)
    v_ref[...] = jnp.dot(x_ref[...], w1b_ref[...],
                         preferred_element_type=jnp.float32)


def _uv(x, w1aT, w1bT):
    nb = 10
    rb = N // nb
    row = lambda b: (b, 0)
    full = lambda b: (0, 0)
    return pl.pallas_call(
        _uv_body,
        grid=(nb,),
        in_specs=[
            pl.BlockSpec((rb, D), row),
            pl.BlockSpec((D, D), full),
            pl.BlockSpec((D, D), full),
        ],
        out_specs=(pl.BlockSpec((rb, D), row), pl.BlockSpec((rb, D), row)),
        out_shape=(jax.ShapeDtypeStruct((N, D), jnp.float32),
                   jax.ShapeDtypeStruct((N, D), jnp.float32)),
    )(x, w1aT, w1bT)


_EB = 3200

_GP = 0.3275911
_GA1 = 0.254829592
_GA2 = -0.284496736
_GA3 = 1.421413741
_GA4 = -1.453152027
_GA5 = 1.061405429


def _gelu_exact(x):
    z = jnp.abs(x) * 0.7071067811865476
    t = 1.0 / (1.0 + _GP * z)
    poly = t * (_GA1 + t * (_GA2 + t * (_GA3 + t * (_GA4 + t * _GA5))))
    e = 1.0 - poly * jnp.exp(-z * z)
    erf = jnp.where(x >= 0, e, -e)
    return 0.5 * x * (1.0 + erf)


def _mlp_body(g_ref, ea_ref, w1c_ref, b1_ref, gm_ref, be_ref, w2_ref, b2_ref,
              o_ref):
    h = g_ref[...] + jnp.dot(ea_ref[...], w1c_ref[...],
                             preferred_element_type=jnp.float32)
    h = h + b1_ref[...]
    mean = jnp.mean(h, axis=1, keepdims=True)
    dlt = h - mean
    var = jnp.mean(dlt * dlt, axis=1, keepdims=True)
    hn = dlt * lax.rsqrt(var + 1e-5)
    hn = hn * gm_ref[...] + be_ref[...]
    ge = _gelu_exact(hn)
    o_ref[...] = jnp.dot(ge, w2_ref[...],
                         preferred_element_type=jnp.float32) + b2_ref[...]


def _mlp(g, ea, w1cT, b1, gamma, beta, w2T, b2):
    ne = g.shape[0]
    nb = ne // _EB
    row = lambda b: (b, 0)
    full = lambda b: (0, 0)
    return pl.pallas_call(
        _mlp_body,
        grid=(nb,),
        in_specs=[
            pl.BlockSpec((_EB, D), row),
            pl.BlockSpec((_EB, DE), row),
            pl.BlockSpec((DE, D), full),
            pl.BlockSpec((1, D), full),
            pl.BlockSpec((1, D), full),
            pl.BlockSpec((1, D), full),
            pl.BlockSpec((D, D), full),
            pl.BlockSpec((1, D), full),
        ],
        out_specs=pl.BlockSpec((_EB, D), row),
        out_shape=jax.ShapeDtypeStruct((ne, D), jnp.float32),
    )(g, ea, w1cT, b1, gamma, beta, w2T, b2)


def _add_body(p1_ref, p2_ref, o_ref):
    o_ref[...] = (p1_ref[0] + p1_ref[1]) + (p2_ref[0] + p2_ref[1])


def _add_partials(p1, p2):
    nb = 10
    rb = N // nb
    spec = pl.BlockSpec((NC, rb, D), lambda b: (0, b, 0))
    return pl.pallas_call(
        _add_body,
        grid=(nb,),
        in_specs=[spec, spec],
        out_specs=pl.BlockSpec((rb, D), lambda b: (b, 0)),
        out_shape=jax.ShapeDtypeStruct((N, D), jnp.float32),
    )(p1, p2)


def kernel(x, edge_index, edge_attr, W1, b1, gamma, beta, W2, b2):
    ii = edge_index[0]
    jj = edge_index[1]
    w1aT = W1[:, :D].T
    w1bT = W1[:, D:2 * D].T
    w1cT = W1[:, 2 * D:].T
    b1r = b1[None, :]
    gammar = gamma[None, :]
    betar = beta[None, :]
    w2T = W2.T
    b2r = b2[None, :]

    u, v = _uv(x, w1aT, w1bT)
    parts = []
    for h in range(SC_SPLIT):
        sl = slice(h * ES, (h + 1) * ES)
        g = _gather_sc(u, v, ii[sl], jj[sl])
        m = _mlp(g, edge_attr[sl], w1cT, b1r, gammar, betar, w2T, b2r)
        parts.append(_scatter_sc(m, jj[sl]))
    return _add_partials(*parts)


# speedup vs baseline: 5.6825x; 1.0294x over previous
import functools

import jax
import jax.numpy as jnp
from jax import lax
from jax.experimental import pallas as pl
from jax.experimental.pallas import tpu as pltpu
import jax.experimental.pallas.tpu_sc as plsc

E = 320000
N = 10000
D = 128
DE = 16

SC_SPLIT = 2
ES = E // SC_SPLIT

NC = 2
NS = 16
NW = NC * NS
C = 40
ZC = 80
NZ = N // ZC
ZPT = (NZ + NS - 1) // NS

_mesh = plsc.VectorSubcoreMesh(core_axis_name="c", subcore_axis_name="s",
                               num_cores=NC, num_subcores=NS)


def _make_gather(ne):
    ew = ne // NW
    chunks = ew // C
    nb4 = 4
    groups = chunks // nb4
    tail = chunks % nb4

    @functools.partial(
        pl.kernel,
        out_type=jax.ShapeDtypeStruct((ne, D), jnp.float32),
        mesh=_mesh,
        scratch_types=[
            pltpu.VMEM((ew,), jnp.int32),
            pltpu.VMEM((ew,), jnp.int32),
            pltpu.VMEM((nb4, C, D), jnp.float32),
            pltpu.VMEM((nb4, C, D), jnp.float32),
            pltpu.SemaphoreType.DMA((nb4,)),
            pltpu.SemaphoreType.DMA((nb4,)),
            pltpu.SemaphoreType.DMA((nb4,)),
        ],
    )
    def gather_kernel(u_hbm, v_hbm, ii_hbm, jj_hbm, g_hbm,
                      iib, jjb, ru, rv, sgu, sgv, sw):
        cid = lax.axis_index("c")
        sid = lax.axis_index("s")
        wid = cid * NS + sid
        base0 = wid * ew

        pltpu.sync_copy(ii_hbm.at[pl.ds(base0, ew)], iib)
        pltpu.sync_copy(jj_hbm.at[pl.ds(base0, ew)], jjb)

        def start_gather(k, b):
            pltpu.async_copy(u_hbm.at[iib.at[pl.ds(k * C, C)]], ru.at[b],
                             sgu.at[b])
            pltpu.async_copy(v_hbm.at[jjb.at[pl.ds(k * C, C)]], rv.at[b],
                             sgv.at[b])

        def wait_gather(b):
            pltpu.make_async_copy(u_hbm.at[pl.ds(0, C)], ru.at[b],
                                  sgu.at[b]).wait()
            pltpu.make_async_copy(v_hbm.at[pl.ds(0, C)], rv.at[b],
                                  sgv.at[b]).wait()

        def add_rows(b):
            @plsc.parallel_loop(0, C, unroll=8)
            def _(e):
                for c8 in range(D // 16):
                    sl = pl.ds(c8 * 16, 16)
                    ru[b, e, sl] = ru[b, e, sl] + rv[b, e, sl]

        def start_writeout(k, b):
            pltpu.async_copy(ru.at[b], g_hbm.at[pl.ds(base0 + k * C, C)],
                             sw.at[b])

        def wait_writeout(b):
            pltpu.make_async_copy(ru.at[b], g_hbm.at[pl.ds(0, C)],
                                  sw.at[b]).wait()

        for b in range(nb4 - 1):
            start_gather(b, b)

        def group(g, carry):
            for b in range(nb4):
                k = nb4 * g + b
                wait_gather(b)

                @pl.when(k + nb4 - 1 < chunks)
                def _():
                    @pl.when(k >= 1)
                    def _():
                        wait_writeout((b + nb4 - 1) % nb4)

                    start_gather(k + nb4 - 1, (b + nb4 - 1) % nb4)

                add_rows(b)
                start_writeout(k, b)
            return carry

        lax.fori_loop(0, groups, group, 0)
        for t in range(tail):
            k = groups * nb4 + t
            b = k % nb4
            wait_gather(b)
            add_rows(b)
            start_writeout(k, b)
        for b in range(nb4):
            wait_writeout(b)

    return gather_kernel


def _make_scatter(ne):
    ew = ne // NW
    chunks = ew // C
    nb4 = 4
    groups = chunks // nb4
    tail = chunks % nb4

    @functools.partial(
        pl.kernel,
        out_type=jax.ShapeDtypeStruct((NC, N, D), jnp.float32),
        mesh=_mesh,
        scratch_types=[
            pltpu.VMEM((nb4, C), jnp.int32),
            pltpu.VMEM((nb4, C, D), jnp.float32),
            pltpu.VMEM((ZC, D), jnp.float32),
            pltpu.VMEM_SHARED((N, D), jnp.float32),
            pltpu.SemaphoreType.DMA((nb4,)),
            pltpu.SemaphoreType.DMA((nb4,)),
            pltpu.SemaphoreType.DMA((nb4,)),
        ],
    )
    def scatter_kernel(m_hbm, jj_hbm, out_hbm, jjb, rows, stg, acc_sh,
                       sji, sri, ssc):
        cid = lax.axis_index("c")
        sid = lax.axis_index("s")

        zero16 = jnp.zeros((16,), jnp.float32)

        @plsc.parallel_loop(0, ZC, unroll=4)
        def _(r):
            for c8 in range(D // 16):
                stg[r, pl.ds(c8 * 16, 16)] = zero16

        for k in range(ZPT):
            ch = sid * ZPT + k

            @pl.when(ch < NZ)
            def _():
                pltpu.sync_copy(stg, acc_sh.at[pl.ds(ch * ZC, ZC)])

        plsc.subcore_barrier()

        base0 = cid * (ne // NC) + sid * ew

        def start_in(k, b):
            pltpu.async_copy(jj_hbm.at[pl.ds(base0 + k * C, C)], jjb.at[b],
                             sji.at[b])
            pltpu.async_copy(m_hbm.at[pl.ds(base0 + k * C, C)], rows.at[b],
                             sri.at[b])

        def wait_in(b):
            pltpu.make_async_copy(jj_hbm.at[pl.ds(0, C)], jjb.at[b],
                                  sji.at[b]).wait()
            pltpu.make_async_copy(m_hbm.at[pl.ds(0, C)], rows.at[b],
                                  sri.at[b]).wait()

        def start_scat(b):
            pltpu.async_copy(rows.at[b], acc_sh.at[jjb.at[b]], ssc.at[b],
                             add=True)

        def wait_scat(b):
            pltpu.make_async_copy(rows.at[b], acc_sh.at[pl.ds(0, C)],
                                  ssc.at[b]).wait()

        for b in range(nb4 - 1):
            start_in(b, b)

        def group(g, carry):
            for b in range(nb4):
                k = nb4 * g + b
                wait_in(b)
                start_scat(b)

                @pl.when(k + nb4 - 1 < chunks)
                def _():
                    @pl.when(k >= 1)
                    def _():
                        wait_scat((b + nb4 - 1) % nb4)

                    start_in(k + nb4 - 1, (b + nb4 - 1) % nb4)
            return carry

        lax.fori_loop(0, groups, group, 0)
        for t in range(tail):
            k = groups * nb4 + t
            b = k % nb4
            wait_in(b)
            start_scat(b)
        for b in range(nb4):
            wait_scat(b)
        plsc.subcore_barrier()

        for k in range(ZPT):
            ch = sid * ZPT + k

            @pl.when(ch < NZ)
            def _():
                pltpu.sync_copy(acc_sh.at[pl.ds(ch * ZC, ZC)], stg)
                pltpu.sync_copy(stg, out_hbm.at[cid, pl.ds(ch * ZC, ZC)])

    return scatter_kernel


_gather_sc = _make_gather(ES)
_scatter_sc = _make_scatter(ES)


def _uv_body(x_ref, w1a_ref, w1b_ref, u_ref, v_ref):
    u_ref[...] = jnp.dot(x_ref[...], w1a_ref[...],
                         preferred_element_type=jnp.float32)
    v_ref[...] = jnp.dot(x_ref[...], w1b_ref[...],
                         preferred_element_type=jnp.float32)


def _uv(x, w1aT, w1bT):
    nb = 10
    rb = N // nb
    row = lambda b: (b, 0)
    full = lambda b: (0, 0)
    return pl.pallas_call(
        _uv_body,
        grid=(nb,),
        in_specs=[
            pl.BlockSpec((rb, D), row),
            pl.BlockSpec((D, D), full),
            pl.BlockSpec((D, D), full),
        ],
        out_specs=(pl.BlockSpec((rb, D), row), pl.BlockSpec((rb, D), row)),
        out_shape=(jax.ShapeDtypeStruct((N, D), jnp.float32),
                   jax.ShapeDtypeStruct((N, D), jnp.float32)),
    )(x, w1aT, w1bT)


_EB = 8000

_GP = 0.3275911
_GA1 = 0.254829592
_GA2 = -0.284496736
_GA3 = 1.421413741
_GA4 = -1.453152027
_GA5 = 1.061405429


def _gelu_exact(x):
    z = jnp.abs(x) * 0.7071067811865476
    t = 1.0 / (1.0 + _GP * z)
    poly = t * (_GA1 + t * (_GA2 + t * (_GA3 + t * (_GA4 + t * _GA5))))
    e = 1.0 - poly * jnp.exp(-z * z)
    erf = jnp.where(x >= 0, e, -e)
    return 0.5 * x * (1.0 + erf)


def _mlp_body(g_ref, ea_ref, w1c_ref, b1_ref, gm_ref, be_ref, w2_ref, b2_ref,
              o_ref):
    h = g_ref[...] + jnp.dot(ea_ref[...], w1c_ref[...],
                             preferred_element_type=jnp.float32)
    h = h + b1_ref[...]
    mean = jnp.mean(h, axis=1, keepdims=True)
    dlt = h - mean
    var = jnp.mean(dlt * dlt, axis=1, keepdims=True)
    hn = dlt * lax.rsqrt(var + 1e-5)
    hn = hn * gm_ref[...] + be_ref[...]
    ge = _gelu_exact(hn)
    o_ref[...] = jnp.dot(ge, w2_ref[...],
                         preferred_element_type=jnp.float32) + b2_ref[...]


def _mlp(g, ea, w1cT, b1, gamma, beta, w2T, b2):
    ne = g.shape[0]
    nb = ne // _EB
    row = lambda b: (b, 0)
    full = lambda b: (0, 0)
    return pl.pallas_call(
        _mlp_body,
        grid=(nb,),
        in_specs=[
            pl.BlockSpec((_EB, D), row),
            pl.BlockSpec((_EB, DE), row),
            pl.BlockSpec((DE, D), full),
            pl.BlockSpec((1, D), full),
            pl.BlockSpec((1, D), full),
            pl.BlockSpec((1, D), full),
            pl.BlockSpec((D, D), full),
            pl.BlockSpec((1, D), full),
        ],
        out_specs=pl.BlockSpec((_EB, D), row),
        out_shape=jax.ShapeDtypeStruct((ne, D), jnp.float32),
    )(g, ea, w1cT, b1, gamma, beta, w2T, b2)


def _add_body(p1_ref, p2_ref, o_ref):
    o_ref[...] = (p1_ref[0] + p1_ref[1]) + (p2_ref[0] + p2_ref[1])


def _add_partials(p1, p2):
    nb = 10
    rb = N // nb
    spec = pl.BlockSpec((NC, rb, D), lambda b: (0, b, 0))
    return pl.pallas_call(
        _add_body,
        grid=(nb,),
        in_specs=[spec, spec],
        out_specs=pl.BlockSpec((rb, D), lambda b: (b, 0)),
        out_shape=jax.ShapeDtypeStruct((N, D), jnp.float32),
    )(p1, p2)


def kernel(x, edge_index, edge_attr, W1, b1, gamma, beta, W2, b2):
    ii = edge_index[0]
    jj = edge_index[1]
    w1aT = W1[:, :D].T
    w1bT = W1[:, D:2 * D].T
    w1cT = W1[:, 2 * D:].T
    b1r = b1[None, :]
    gammar = gamma[None, :]
    betar = beta[None, :]
    w2T = W2.T
    b2r = b2[None, :]

    u, v = _uv(x, w1aT, w1bT)
    parts = []
    for h in range(SC_SPLIT):
        sl = slice(h * ES, (h + 1) * ES)
        g = _gather_sc(u, v, ii[sl], jj[sl])
        m = _mlp(g, edge_attr[sl], w1cT, b1r, gammar, betar, w2T, b2r)
        parts.append(_scatter_sc(m, jj[sl]))
    return _add_partials(*parts)
